# baseline (speedup 1.0000x reference)
import functools

import jax
import jax.numpy as jnp
from jax import lax
from jax.experimental import pallas as pl
from jax.experimental.pallas import tpu as pltpu
from jax.experimental.pallas import tpu_sc as plsc

N = 10000
E = 320000
D = 128
HD = 128
L = 16
NC = 2
NS = 16
NW = NC * NS
EW = E // NW
CHUNK = 80
NCHUNK = EW // CHUNK
CW = 80
NROWC = N // CW
C3 = 2000
GW = 80
NB = 1000
EB = 8000


def _mesh():
    return plsc.VectorSubcoreMesh(core_axis_name="c", subcore_axis_name="s")


def _node_mlp_body(x_ref, w1_ref, b1_ref, wpq_ref, bc1_ref, p_ref, q_ref):
    xb = x_ref[...]
    h = jnp.tanh(jnp.dot(xb, w1_ref[...], preferred_element_type=jnp.float32)
                 + b1_ref[...])
    xc = jnp.concatenate([h, xb], axis=1)
    pq = jnp.dot(xc, wpq_ref[...], preferred_element_type=jnp.float32)
    p_ref[...] = pq[:, :HD] + bc1_ref[...]
    q_ref[...] = pq[:, HD:]


def _node_mlp(x, W1, b1r, Wpq, bc1r):
    return pl.pallas_call(
        _node_mlp_body,
        grid=(N // NB,),
        in_specs=[
            pl.BlockSpec((NB, D), lambda i: (i, 0)),
            pl.BlockSpec((D, HD), lambda i: (0, 0)),
            pl.BlockSpec((1, HD), lambda i: (0, 0)),
            pl.BlockSpec((HD + D, 2 * HD), lambda i: (0, 0)),
            pl.BlockSpec((1, HD), lambda i: (0, 0)),
        ],
        out_specs=[
            pl.BlockSpec((NB, HD), lambda i: (i, 0)),
            pl.BlockSpec((NB, HD), lambda i: (i, 0)),
        ],
        out_shape=[
            jax.ShapeDtypeStruct((N, HD), jnp.float32),
            jax.ShapeDtypeStruct((N, HD), jnp.float32),
        ],
    )(x, W1, b1r, Wpq, bc1r)


HDP = HD // 2


@functools.partial(
    pl.kernel,
    out_type=[jax.ShapeDtypeStruct((N, HDP), jnp.int32),
              jax.ShapeDtypeStruct((N, HDP), jnp.int32)],
    mesh=_mesh(),
    compiler_params=pltpu.CompilerParams(use_tc_tiling_on_sc=False),
    scratch_types=[
        pltpu.VMEM((CW, HD), jnp.float32),
        pltpu.VMEM((CW, HDP), jnp.int32),
    ],
)
def _pack_tables(p_hbm, q_hbm, ppk_hbm, qpk_hbm, fbuf, pkbuf):
    wid = lax.axis_index("s") * NC + lax.axis_index("c")
    for tbl_hbm, pk_hbm in ((p_hbm, ppk_hbm), (q_hbm, qpk_hbm)):
        for k in range(-(-NROWC // NW)):
            c = wid + k * NW
            @pl.when(c < NROWC)
            def _():
                pltpu.sync_copy(tbl_hbm.at[pl.ds(c * CW, CW)], fbuf)

                @plsc.parallel_loop(0, CW, unroll=2)
                def row(r):
                    for k2 in range(HD // (2 * L)):
                        a = lax.bitcast_convert_type(
                            fbuf[r, pl.ds(k2 * 2 * L, L)], jnp.int32)
                        b = lax.bitcast_convert_type(
                            fbuf[r, pl.ds(k2 * 2 * L + L, L)], jnp.int32)
                        ar = a + 0x7FFF + ((a >> 16) & 1)
                        br = b + 0x7FFF + ((b >> 16) & 1)
                        pkbuf[r, pl.ds(k2 * L, L)] = (
                            ((ar >> 16) & 0xFFFF) | (br & jnp.int32(-65536)))
                pltpu.sync_copy(pkbuf, pk_hbm.at[pl.ds(c * CW, CW)])


BLK = 2000
SUB = 80
NSUB = BLK // SUB
E0 = 192000
E1 = E - E0


@functools.lru_cache(maxsize=None)
def _make_gather_relu(e_cnt, e_lo):
    ew = e_cnt // NW
    nblk = ew // BLK

    @functools.partial(
        pl.kernel,
        out_type=jax.ShapeDtypeStruct((e_cnt, HD), jnp.float32),
        mesh=_mesh(),
        compiler_params=pltpu.CompilerParams(use_tc_tiling_on_sc=False),
        scratch_types=[
            pltpu.VMEM((BLK,), jnp.int32),
            pltpu.VMEM((BLK,), jnp.int32),
            pltpu.VMEM((SUB, HDP), jnp.int32),
            pltpu.VMEM((SUB, HDP), jnp.int32),
            pltpu.VMEM((SUB, HDP), jnp.int32),
            pltpu.VMEM((SUB, HDP), jnp.int32),
            pltpu.VMEM((SUB, HD), jnp.float32),
            pltpu.VMEM((SUB, HD), jnp.float32),
            pltpu.SemaphoreType.DMA,
            pltpu.SemaphoreType.DMA,
            pltpu.SemaphoreType.DMA,
            pltpu.SemaphoreType.DMA,
        ],
    )
    def gather_relu(p_hbm, q_hbm, dst_hbm, src_hbm, t_hbm,
                    dst_v, src_v, pg0, pg1, qg0, qg1, t0, t1,
                    gsem0, gsem1, wsem0, wsem1):
        sid = lax.axis_index("s")
        wid = sid * NC + lax.axis_index("c")
        base = wid * ew
        pgs, qgs, ts = (pg0, pg1), (qg0, qg1), (t0, t1)
        gsems, wsems = (gsem0, gsem1), (wsem0, wsem1)

        def fire(s, buf):
            sl = pl.ds(s * SUB, SUB)
            cp = pltpu.async_copy(p_hbm.at[dst_v.at[sl]], pgs[buf],
                                  gsems[buf])
            cq = pltpu.async_copy(q_hbm.at[src_v.at[sl]], qgs[buf],
                                  gsems[buf])
            return cp, cq

        def block(blk, carry):
            off = base + blk * BLK
            pltpu.sync_copy(dst_hbm.at[pl.ds(e_lo + off, BLK)], dst_v)
            pltpu.sync_copy(src_hbm.at[pl.ds(e_lo + off, BLK)], src_v)
            gath = {0: fire(0, 0)}
            wb = {}
            for s in range(NSUB):
                cur = s & 1
                if s + 1 < NSUB:
                    gath[s + 1] = fire(s + 1, 1 - cur)
                for cp in gath.pop(s):
                    cp.wait()
                if s >= 2:
                    wb.pop(s - 2).wait()
                pg_v, qg_v, t_v = pgs[cur], qgs[cur], ts[cur]

                @plsc.parallel_loop(0, SUB, unroll=2)
                def row(r):
                    for k in range(HD // (2 * L)):
                        pi = pg_v[r, pl.ds(k * L, L)]
                        qi = qg_v[r, pl.ds(k * L, L)]
                        pa = lax.bitcast_convert_type(pi << 16, jnp.float32)
                        pb = lax.bitcast_convert_type(
                            pi & jnp.int32(-65536), jnp.float32)
                        qa = lax.bitcast_convert_type(qi << 16, jnp.float32)
                        qb = lax.bitcast_convert_type(
                            qi & jnp.int32(-65536), jnp.float32)
                        t_v[r, pl.ds(k * 2 * L, L)] = jnp.maximum(
                            pa + qa, 0.0)
                        t_v[r, pl.ds(k * 2 * L + L, L)] = jnp.maximum(
                            pb + qb, 0.0)
                wb[s] = pltpu.async_copy(
                    t_v, t_hbm.at[pl.ds(off + s * SUB, SUB)], wsems[cur])
            for s in sorted(wb):
                wb[s].wait()
            return carry

        lax.fori_loop(0, nblk, block, 0)

    return gather_relu


def _edge_mlp_body(t_ref, w_ref, b_ref, m_ref):
    acc = jnp.dot(t_ref[...], w_ref[...], preferred_element_type=jnp.float32)
    m_ref[...] = jnp.maximum(acc + b_ref[...], 0.0)


def _edge_mlp(t, Wc2, bc2r):
    ne = t.shape[0]
    return pl.pallas_call(
        _edge_mlp_body,
        grid=(ne // EB,),
        in_specs=[
            pl.BlockSpec((EB, HD), lambda i: (i, 0)),
            pl.BlockSpec((HD, HD), lambda i: (0, 0)),
            pl.BlockSpec((1, HD), lambda i: (0, 0)),
        ],
        out_specs=pl.BlockSpec((EB, HD), lambda i: (i, 0)),
        out_shape=jax.ShapeDtypeStruct((ne, HD), jnp.float32),
    )(t, Wc2, bc2r)


@functools.lru_cache(maxsize=None)
def _make_scatter_add(e_cnt, e_lo):
    ew = e_cnt // NW

    @functools.partial(
        pl.kernel,
        out_type=jax.ShapeDtypeStruct((NC, N, HD), jnp.float32),
        mesh=_mesh(),
        scratch_types=[
            pltpu.VMEM((CHUNK,), jnp.int32),
            pltpu.VMEM((CHUNK,), jnp.int32),
            pltpu.VMEM((CHUNK, HD), jnp.float32),
            pltpu.VMEM((CHUNK, HD), jnp.float32),
            pltpu.VMEM((CW, HD), jnp.float32),
            pltpu.VMEM_SHARED((N, HD), jnp.float32),
            pltpu.SemaphoreType.DMA,
            pltpu.SemaphoreType.DMA,
        ],
    )
    def scatter_add(m_hbm, dst_hbm, out_hbm, idx0, idx1, rows0, rows1,
                    zero_v, h2_sh, msem0, msem1):
        cid = lax.axis_index("c")
        sid = lax.axis_index("s")
        wid = sid * NC + cid

        def zrow(r, carry):
            for k in range(HD // L):
                zero_v[r, pl.ds(k * L, L)] = jnp.zeros((L,), jnp.float32)
            return carry

        lax.fori_loop(0, CW, zrow, 0)
        for k in range(-(-NROWC // NS)):
            c = sid + k * NS
            @pl.when(c < NROWC)
            def _():
                pltpu.sync_copy(zero_v, h2_sh.at[pl.ds(c * CW, CW)])
        plsc.subcore_barrier()

        idxs, rows, msems = (idx0, idx1), (rows0, rows1), (msem0, msem1)
        base = wid * ew

        def fire(off, s, buf):
            sl = pl.ds(off + s * CHUNK, CHUNK)
            cm = pltpu.async_copy(m_hbm.at[sl], rows[buf], msems[buf])
            ci = pltpu.async_copy(dst_hbm.at[pl.ds(e_lo + off + s * CHUNK,
                                                   CHUNK)],
                                  idxs[buf], msems[buf])
            return cm, ci

        def block(blk, carry):
            off = base + blk * BLK
            pend = {0: fire(off, 0, 0)}
            for s in range(NSUB):
                cur = s & 1
                if s + 1 < NSUB:
                    pend[s + 1] = fire(off, s + 1, 1 - cur)
                for cp in pend.pop(s):
                    cp.wait()
                pltpu.sync_copy(rows[cur], h2_sh.at[idxs[cur]], add=True)
            return carry

        lax.fori_loop(0, ew // BLK, block, 0)
        plsc.subcore_barrier()

        for k in range(-(-NROWC // NS)):
            c = sid + k * NS
            @pl.when(c < NROWC)
            def _():
                pltpu.sync_copy(h2_sh.at[pl.ds(c * CW, CW)],
                                out_hbm.at[cid, pl.ds(c * CW, CW)])

    return scatter_add


def _combine_body(part0_ref, part1_ref, x_ref, weh_ref, wex_ref, c_ref,
                  ab_ref):
    h2 = (part0_ref[0] + part0_ref[1]) + (part1_ref[0] + part1_ref[1])
    ab = (jnp.dot(h2, weh_ref[...], preferred_element_type=jnp.float32)
          + jnp.dot(x_ref[...], wex_ref[...],
                    preferred_element_type=jnp.float32)
          + c_ref[...])
    ab_ref[...] = ab


def _combine(parts0, parts1, x, WeH, WeX, cvec):
    return pl.pallas_call(
        _combine_body,
        grid=(N // NB,),
        in_specs=[
            pl.BlockSpec((NC, NB, HD), lambda i: (0, i, 0)),
            pl.BlockSpec((NC, NB, HD), lambda i: (0, i, 0)),
            pl.BlockSpec((NB, D), lambda i: (i, 0)),
            pl.BlockSpec((HD, 128), lambda i: (0, 0)),
            pl.BlockSpec((D, 128), lambda i: (0, 0)),
            pl.BlockSpec((1, 128), lambda i: (0, 0)),
        ],
        out_specs=pl.BlockSpec((NB, 128), lambda i: (i, 0)),
        out_shape=jax.ShapeDtypeStruct((N, 128), jnp.float32),
    )(parts0, parts1, x, WeH, WeX, cvec)


@functools.partial(
    pl.kernel,
    out_type=jax.ShapeDtypeStruct((E,), jnp.float32),
    mesh=_mesh(),
    scratch_types=[
        pltpu.VMEM((C3,), jnp.int32),
        pltpu.VMEM((C3,), jnp.int32),
        pltpu.VMEM((C3,), jnp.float32),
        pltpu.VMEM((C3,), jnp.float32),
        pltpu.VMEM((C3,), jnp.float32),
        pltpu.SemaphoreType.DMA,
        pltpu.SemaphoreType.DMA,
    ],
)
def _edge_out(a_hbm, b_hbm, src_hbm, dst_hbm, out_hbm,
              src_v, dst_v, ag_v, bg_v, o_v, sem1, sem2):
    wid = lax.axis_index("s") * NC + lax.axis_index("c")
    base = wid * EW

    def chunk(i, carry):
        off = base + i * C3
        pltpu.sync_copy(src_hbm.at[pl.ds(off, C3)], src_v)
        pltpu.sync_copy(dst_hbm.at[pl.ds(off, C3)], dst_v)
        cps = []
        for g in range(C3 // GW):
            sl = pl.ds(g * GW, GW)
            cps.append(pltpu.async_copy(a_hbm.at[src_v.at[sl]],
                                        ag_v.at[sl], sem1))
            cps.append(pltpu.async_copy(b_hbm.at[dst_v.at[sl]],
                                        bg_v.at[sl], sem2))
        for cp in cps:
            cp.wait()

        def vec(j, c2):
            sl = pl.ds(j * L, L)
            z = ag_v[sl] + bg_v[sl]
            o_v[sl] = 1.0 / (1.0 + jnp.exp(-z))
            return c2

        lax.fori_loop(0, C3 // L, vec, 0)
        pltpu.sync_copy(o_v, out_hbm.at[pl.ds(off, C3)])
        return carry

    lax.fori_loop(0, EW // C3, chunk, 0)


def kernel(x, edge_index, W1, b1, Wc1, bc1, Wc2, bc2, We, be):
    src = edge_index[0]
    dst = edge_index[1]

    U = Wc1[:HD + D]
    V = Wc1[HD + D:]
    Wpq = jnp.concatenate([U - V, V], axis=1)

    P, Q = _node_mlp(x, W1, b1.reshape(1, HD), Wpq, bc1.reshape(1, HD))
    bc2r = bc2.reshape(1, HD)
    Ppk, Qpk = _pack_tables(P, Q)
    t0 = _make_gather_relu(E0, 0)(Ppk, Qpk, dst, src)
    m0 = _edge_mlp(t0, Wc2, bc2r)
    t1 = _make_gather_relu(E1, E0)(Ppk, Qpk, dst, src)
    m1 = _edge_mlp(t1, Wc2, bc2r)
    parts0 = _make_scatter_add(E0, 0)(m0, dst)
    parts1 = _make_scatter_add(E1, E0)(m1, dst)

    we = We[:, 0]
    WeH = jnp.zeros((HD, 128), jnp.float32)
    WeH = WeH.at[:, 0].set(we[:HD]).at[:, 1].set(we[2 * HD:3 * HD])
    WeX = jnp.zeros((D, 128), jnp.float32)
    WeX = WeX.at[:, 0].set(we[HD:2 * HD]).at[:, 1].set(we[3 * HD:])
    cvec = jnp.zeros((1, 128), jnp.float32).at[0, 0].set(be[0])

    ab = _combine(parts0, parts1, x, WeH, WeX, cvec)
    a = ab[:, 0]
    b = ab[:, 1]
    return _edge_out(a, b, src, dst)


# speedup vs baseline: 1.0030x; 1.0030x over previous
import functools

import jax
import jax.numpy as jnp
from jax import lax
from jax.experimental import pallas as pl
from jax.experimental.pallas import tpu as pltpu
from jax.experimental.pallas import tpu_sc as plsc

N = 10000
E = 320000
D = 128
HD = 128
L = 16
NC = 2
NS = 16
NW = NC * NS
EW = E // NW
CHUNK = 80
NCHUNK = EW // CHUNK
CW = 80
NROWC = N // CW
C3 = 2000
GW = 80
NB = 1000
EB = 4000


def _mesh():
    return plsc.VectorSubcoreMesh(core_axis_name="c", subcore_axis_name="s")


def _node_mlp_body(x_ref, w1_ref, b1_ref, wpq_ref, bc1_ref, p_ref, q_ref):
    xb = x_ref[...]
    h = jnp.tanh(jnp.dot(xb, w1_ref[...], preferred_element_type=jnp.float32)
                 + b1_ref[...])
    xc = jnp.concatenate([h, xb], axis=1)
    pq = jnp.dot(xc, wpq_ref[...], preferred_element_type=jnp.float32)
    p_ref[...] = pq[:, :HD] + bc1_ref[...]
    q_ref[...] = pq[:, HD:]


def _node_mlp(x, W1, b1r, Wpq, bc1r):
    return pl.pallas_call(
        _node_mlp_body,
        grid=(N // NB,),
        in_specs=[
            pl.BlockSpec((NB, D), lambda i: (i, 0)),
            pl.BlockSpec((D, HD), lambda i: (0, 0)),
            pl.BlockSpec((1, HD), lambda i: (0, 0)),
            pl.BlockSpec((HD + D, 2 * HD), lambda i: (0, 0)),
            pl.BlockSpec((1, HD), lambda i: (0, 0)),
        ],
        out_specs=[
            pl.BlockSpec((NB, HD), lambda i: (i, 0)),
            pl.BlockSpec((NB, HD), lambda i: (i, 0)),
        ],
        out_shape=[
            jax.ShapeDtypeStruct((N, HD), jnp.float32),
            jax.ShapeDtypeStruct((N, HD), jnp.float32),
        ],
    )(x, W1, b1r, Wpq, bc1r)


HDP = HD // 2


@functools.partial(
    pl.kernel,
    out_type=[jax.ShapeDtypeStruct((N, HDP), jnp.int32),
              jax.ShapeDtypeStruct((N, HDP), jnp.int32)],
    mesh=_mesh(),
    compiler_params=pltpu.CompilerParams(use_tc_tiling_on_sc=False),
    scratch_types=[
        pltpu.VMEM((CW, HD), jnp.float32),
        pltpu.VMEM((CW, HDP), jnp.int32),
    ],
)
def _pack_tables(p_hbm, q_hbm, ppk_hbm, qpk_hbm, fbuf, pkbuf):
    wid = lax.axis_index("s") * NC + lax.axis_index("c")
    for tbl_hbm, pk_hbm in ((p_hbm, ppk_hbm), (q_hbm, qpk_hbm)):
        for k in range(-(-NROWC // NW)):
            c = wid + k * NW
            @pl.when(c < NROWC)
            def _():
                pltpu.sync_copy(tbl_hbm.at[pl.ds(c * CW, CW)], fbuf)

                @plsc.parallel_loop(0, CW, unroll=2)
                def row(r):
                    for k2 in range(HD // (2 * L)):
                        a = lax.bitcast_convert_type(
                            fbuf[r, pl.ds(k2 * 2 * L, L)], jnp.int32)
                        b = lax.bitcast_convert_type(
                            fbuf[r, pl.ds(k2 * 2 * L + L, L)], jnp.int32)
                        ar = a + 0x7FFF + ((a >> 16) & 1)
                        br = b + 0x7FFF + ((b >> 16) & 1)
                        pkbuf[r, pl.ds(k2 * L, L)] = (
                            ((ar >> 16) & 0xFFFF) | (br & jnp.int32(-65536)))
                pltpu.sync_copy(pkbuf, pk_hbm.at[pl.ds(c * CW, CW)])


BLK = 2000
SUB = 80
NSUB = BLK // SUB
E0 = 192000
E1 = E - E0


@functools.lru_cache(maxsize=None)
def _make_gather_relu(e_cnt, e_lo):
    ew = e_cnt // NW
    nblk = ew // BLK

    @functools.partial(
        pl.kernel,
        out_type=jax.ShapeDtypeStruct((e_cnt, HD), jnp.float32),
        mesh=_mesh(),
        compiler_params=pltpu.CompilerParams(use_tc_tiling_on_sc=False),
        scratch_types=[
            pltpu.VMEM((BLK,), jnp.int32),
            pltpu.VMEM((BLK,), jnp.int32),
            pltpu.VMEM((SUB, HDP), jnp.int32),
            pltpu.VMEM((SUB, HDP), jnp.int32),
            pltpu.VMEM((SUB, HDP), jnp.int32),
            pltpu.VMEM((SUB, HDP), jnp.int32),
            pltpu.VMEM((SUB, HD), jnp.float32),
            pltpu.VMEM((SUB, HD), jnp.float32),
            pltpu.SemaphoreType.DMA,
            pltpu.SemaphoreType.DMA,
            pltpu.SemaphoreType.DMA,
            pltpu.SemaphoreType.DMA,
        ],
    )
    def gather_relu(p_hbm, q_hbm, dst_hbm, src_hbm, t_hbm,
                    dst_v, src_v, pg0, pg1, qg0, qg1, t0, t1,
                    gsem0, gsem1, wsem0, wsem1):
        sid = lax.axis_index("s")
        wid = sid * NC + lax.axis_index("c")
        base = wid * ew
        pgs, qgs, ts = (pg0, pg1), (qg0, qg1), (t0, t1)
        gsems, wsems = (gsem0, gsem1), (wsem0, wsem1)

        def fire(s, buf):
            sl = pl.ds(s * SUB, SUB)
            cp = pltpu.async_copy(p_hbm.at[dst_v.at[sl]], pgs[buf],
                                  gsems[buf])
            cq = pltpu.async_copy(q_hbm.at[src_v.at[sl]], qgs[buf],
                                  gsems[buf])
            return cp, cq

        def block(blk, carry):
            off = base + blk * BLK
            pltpu.sync_copy(dst_hbm.at[pl.ds(e_lo + off, BLK)], dst_v)
            pltpu.sync_copy(src_hbm.at[pl.ds(e_lo + off, BLK)], src_v)
            gath = {0: fire(0, 0)}
            wb = {}
            for s in range(NSUB):
                cur = s & 1
                if s + 1 < NSUB:
                    gath[s + 1] = fire(s + 1, 1 - cur)
                for cp in gath.pop(s):
                    cp.wait()
                if s >= 2:
                    wb.pop(s - 2).wait()
                pg_v, qg_v, t_v = pgs[cur], qgs[cur], ts[cur]

                @plsc.parallel_loop(0, SUB, unroll=2)
                def row(r):
                    for k in range(HD // (2 * L)):
                        pi = pg_v[r, pl.ds(k * L, L)]
                        qi = qg_v[r, pl.ds(k * L, L)]
                        pa = lax.bitcast_convert_type(pi << 16, jnp.float32)
                        pb = lax.bitcast_convert_type(
                            pi & jnp.int32(-65536), jnp.float32)
                        qa = lax.bitcast_convert_type(qi << 16, jnp.float32)
                        qb = lax.bitcast_convert_type(
                            qi & jnp.int32(-65536), jnp.float32)
                        t_v[r, pl.ds(k * 2 * L, L)] = jnp.maximum(
                            pa + qa, 0.0)
                        t_v[r, pl.ds(k * 2 * L + L, L)] = jnp.maximum(
                            pb + qb, 0.0)
                wb[s] = pltpu.async_copy(
                    t_v, t_hbm.at[pl.ds(off + s * SUB, SUB)], wsems[cur])
            for s in sorted(wb):
                wb[s].wait()
            return carry

        lax.fori_loop(0, nblk, block, 0)

    return gather_relu


def _edge_mlp_body(t_ref, w_ref, b_ref, m_ref):
    acc = jnp.dot(t_ref[...], w_ref[...], preferred_element_type=jnp.float32)
    m_ref[...] = jnp.maximum(acc + b_ref[...], 0.0)


def _edge_mlp(t, Wc2, bc2r):
    ne = t.shape[0]
    return pl.pallas_call(
        _edge_mlp_body,
        grid=(ne // EB,),
        in_specs=[
            pl.BlockSpec((EB, HD), lambda i: (i, 0)),
            pl.BlockSpec((HD, HD), lambda i: (0, 0)),
            pl.BlockSpec((1, HD), lambda i: (0, 0)),
        ],
        out_specs=pl.BlockSpec((EB, HD), lambda i: (i, 0)),
        out_shape=jax.ShapeDtypeStruct((ne, HD), jnp.float32),
    )(t, Wc2, bc2r)


@functools.lru_cache(maxsize=None)
def _make_scatter_add(e_cnt, e_lo):
    ew = e_cnt // NW

    @functools.partial(
        pl.kernel,
        out_type=jax.ShapeDtypeStruct((NC, N, HD), jnp.float32),
        mesh=_mesh(),
        scratch_types=[
            pltpu.VMEM((CHUNK,), jnp.int32),
            pltpu.VMEM((CHUNK,), jnp.int32),
            pltpu.VMEM((CHUNK, HD), jnp.float32),
            pltpu.VMEM((CHUNK, HD), jnp.float32),
            pltpu.VMEM((CW, HD), jnp.float32),
            pltpu.VMEM_SHARED((N, HD), jnp.float32),
            pltpu.SemaphoreType.DMA,
            pltpu.SemaphoreType.DMA,
        ],
    )
    def scatter_add(m_hbm, dst_hbm, out_hbm, idx0, idx1, rows0, rows1,
                    zero_v, h2_sh, msem0, msem1):
        cid = lax.axis_index("c")
        sid = lax.axis_index("s")
        wid = sid * NC + cid

        def zrow(r, carry):
            for k in range(HD // L):
                zero_v[r, pl.ds(k * L, L)] = jnp.zeros((L,), jnp.float32)
            return carry

        lax.fori_loop(0, CW, zrow, 0)
        for k in range(-(-NROWC // NS)):
            c = sid + k * NS
            @pl.when(c < NROWC)
            def _():
                pltpu.sync_copy(zero_v, h2_sh.at[pl.ds(c * CW, CW)])
        plsc.subcore_barrier()

        idxs, rows, msems = (idx0, idx1), (rows0, rows1), (msem0, msem1)
        base = wid * ew

        def fire(off, s, buf):
            sl = pl.ds(off + s * CHUNK, CHUNK)
            cm = pltpu.async_copy(m_hbm.at[sl], rows[buf], msems[buf])
            ci = pltpu.async_copy(dst_hbm.at[pl.ds(e_lo + off + s * CHUNK,
                                                   CHUNK)],
                                  idxs[buf], msems[buf])
            return cm, ci

        def block(blk, carry):
            off = base + blk * BLK
            pend = {0: fire(off, 0, 0)}
            for s in range(NSUB):
                cur = s & 1
                if s + 1 < NSUB:
                    pend[s + 1] = fire(off, s + 1, 1 - cur)
                for cp in pend.pop(s):
                    cp.wait()
                pltpu.sync_copy(rows[cur], h2_sh.at[idxs[cur]], add=True)
            return carry

        lax.fori_loop(0, ew // BLK, block, 0)
        plsc.subcore_barrier()

        for k in range(-(-NROWC // NS)):
            c = sid + k * NS
            @pl.when(c < NROWC)
            def _():
                pltpu.sync_copy(h2_sh.at[pl.ds(c * CW, CW)],
                                out_hbm.at[cid, pl.ds(c * CW, CW)])

    return scatter_add


def _combine_body(part0_ref, part1_ref, x_ref, weh_ref, wex_ref, c_ref,
                  ab_ref):
    h2 = (part0_ref[0] + part0_ref[1]) + (part1_ref[0] + part1_ref[1])
    ab = (jnp.dot(h2, weh_ref[...], preferred_element_type=jnp.float32)
          + jnp.dot(x_ref[...], wex_ref[...],
                    preferred_element_type=jnp.float32)
          + c_ref[...])
    ab_ref[...] = ab


def _combine(parts0, parts1, x, WeH, WeX, cvec):
    return pl.pallas_call(
        _combine_body,
        grid=(N // NB,),
        in_specs=[
            pl.BlockSpec((NC, NB, HD), lambda i: (0, i, 0)),
            pl.BlockSpec((NC, NB, HD), lambda i: (0, i, 0)),
            pl.BlockSpec((NB, D), lambda i: (i, 0)),
            pl.BlockSpec((HD, 128), lambda i: (0, 0)),
            pl.BlockSpec((D, 128), lambda i: (0, 0)),
            pl.BlockSpec((1, 128), lambda i: (0, 0)),
        ],
        out_specs=pl.BlockSpec((NB, 128), lambda i: (i, 0)),
        out_shape=jax.ShapeDtypeStruct((N, 128), jnp.float32),
    )(parts0, parts1, x, WeH, WeX, cvec)


@functools.partial(
    pl.kernel,
    out_type=jax.ShapeDtypeStruct((E,), jnp.float32),
    mesh=_mesh(),
    scratch_types=[
        pltpu.VMEM((C3,), jnp.int32),
        pltpu.VMEM((C3,), jnp.int32),
        pltpu.VMEM((C3,), jnp.float32),
        pltpu.VMEM((C3,), jnp.float32),
        pltpu.VMEM((C3,), jnp.float32),
        pltpu.SemaphoreType.DMA,
        pltpu.SemaphoreType.DMA,
    ],
)
def _edge_out(a_hbm, b_hbm, src_hbm, dst_hbm, out_hbm,
              src_v, dst_v, ag_v, bg_v, o_v, sem1, sem2):
    wid = lax.axis_index("s") * NC + lax.axis_index("c")
    base = wid * EW

    def chunk(i, carry):
        off = base + i * C3
        pltpu.sync_copy(src_hbm.at[pl.ds(off, C3)], src_v)
        pltpu.sync_copy(dst_hbm.at[pl.ds(off, C3)], dst_v)
        cps = []
        for g in range(C3 // GW):
            sl = pl.ds(g * GW, GW)
            cps.append(pltpu.async_copy(a_hbm.at[src_v.at[sl]],
                                        ag_v.at[sl], sem1))
            cps.append(pltpu.async_copy(b_hbm.at[dst_v.at[sl]],
                                        bg_v.at[sl], sem2))
        for cp in cps:
            cp.wait()

        def vec(j, c2):
            sl = pl.ds(j * L, L)
            z = ag_v[sl] + bg_v[sl]
            o_v[sl] = 1.0 / (1.0 + jnp.exp(-z))
            return c2

        lax.fori_loop(0, C3 // L, vec, 0)
        pltpu.sync_copy(o_v, out_hbm.at[pl.ds(off, C3)])
        return carry

    lax.fori_loop(0, EW // C3, chunk, 0)


def kernel(x, edge_index, W1, b1, Wc1, bc1, Wc2, bc2, We, be):
    src = edge_index[0]
    dst = edge_index[1]

    U = Wc1[:HD + D]
    V = Wc1[HD + D:]
    Wpq = jnp.concatenate([U - V, V], axis=1)

    P, Q = _node_mlp(x, W1, b1.reshape(1, HD), Wpq, bc1.reshape(1, HD))
    bc2r = bc2.reshape(1, HD)
    Ppk, Qpk = _pack_tables(P, Q)
    t0 = _make_gather_relu(E0, 0)(Ppk, Qpk, dst, src)
    m0 = _edge_mlp(t0, Wc2, bc2r)
    t1 = _make_gather_relu(E1, E0)(Ppk, Qpk, dst, src)
    m1 = _edge_mlp(t1, Wc2, bc2r)
    parts0 = _make_scatter_add(E0, 0)(m0, dst)
    parts1 = _make_scatter_add(E1, E0)(m1, dst)

    we = We[:, 0]
    WeH = jnp.zeros((HD, 128), jnp.float32)
    WeH = WeH.at[:, 0].set(we[:HD]).at[:, 1].set(we[2 * HD:3 * HD])
    WeX = jnp.zeros((D, 128), jnp.float32)
    WeX = WeX.at[:, 0].set(we[HD:2 * HD]).at[:, 1].set(we[3 * HD:])
    cvec = jnp.zeros((1, 128), jnp.float32).at[0, 0].set(be[0])

    ab = _combine(parts0, parts1, x, WeH, WeX, cvec)
    a = ab[:, 0]
    b = ab[:, 1]
    return _edge_out(a, b, src, dst)
